# Initial kernel scaffold; baseline (speedup 1.0000x reference)
import jax
import jax.numpy as jnp
from jax.experimental import pallas as pl


def kernel(x, edge_index, batch, params):
    raise NotImplementedError("write your pallas kernel here")


# speedup vs baseline: 7.4013x; 7.4013x over previous
import functools

import jax
import jax.numpy as jnp
from jax import lax
from jax.experimental import pallas as pl
from jax.experimental.pallas import tpu as pltpu
from jax.experimental.pallas import tpu_sc as plsc

_N = 100000
_E = 1600000
_F = 5
_G = 1024

_NP = 100352
_ROWS_PER_TILE = 6272
_EROWS = 12544
_EP = _EROWS * 128
_TILE_EROWS = 784
_EITERS = 98
_BLK = 512
_NBLKS = _NP // _BLK
_PG = 1152

@functools.cache
def _sc_mesh():
    return plsc.VectorSubcoreMesh(core_axis_name="c", subcore_axis_name="s")


def _make_agg(ngroups):
    n_pass = (ngroups + 1) // 2
    out_type = [jax.ShapeDtypeStruct((_NP, 16), jnp.float32)] * ngroups
    scratch = [
        pltpu.VMEM((8, 128), jnp.int32),
        pltpu.VMEM((8, 128), jnp.int32),
        pltpu.VMEM((8, 128, 16), jnp.float32),
        pltpu.VMEM((196, 16), jnp.float32),
        pltpu.VMEM_SHARED((_NP, 16), jnp.float32),
        pltpu.SemaphoreType.DMA,
    ]

    @functools.partial(pl.kernel, out_type=out_type, mesh=_sc_mesh(),
                       scratch_types=scratch,
                       compiler_params=pltpu.CompilerParams(
                           use_tc_tiling_on_sc=False))
    def agg(src_hbm, dst_hbm, *rest):
        tables = rest[:ngroups]
        outs = rest[ngroups:2 * ngroups]
        src_v, dst_v, rows, zbuf, shared, sem = rest[2 * ngroups:]
        c = lax.axis_index("c")
        s = lax.axis_index("s")
        base = s * _ROWS_PER_TILE

        def zb(i, carry):
            zbuf[i, :] = jnp.zeros((16,), jnp.float32)
            return carry
        lax.fori_loop(0, 196, zb, 0)

        for p in range(n_pass):
            def zrow(blk, carry):
                pltpu.sync_copy(zbuf,
                                shared.at[pl.ds(base + blk * 196, 196), :])
                return carry
            lax.fori_loop(0, 32, zrow, 0)
            plsc.subcore_barrier()

            for cv in range(2):
                g = 2 * p + cv
                if g >= ngroups:
                    continue

                @pl.when(c == cv)
                def _(g=g):
                    def body(j, carry):
                        row0 = s * _TILE_EROWS + j * 8
                        pltpu.sync_copy(src_hbm.at[pl.ds(row0, 8), :], src_v)
                        pltpu.sync_copy(dst_hbm.at[pl.ds(row0, 8), :], dst_v)
                        cps = [pltpu.async_copy(tables[g].at[src_v.at[k]],
                                                rows.at[k], sem)
                               for k in range(8)]
                        for cp in cps:
                            cp.wait()
                        for k in range(8):
                            pltpu.sync_copy(rows.at[k],
                                            shared.at[dst_v.at[k]], add=True)
                        return carry
                    lax.fori_loop(0, _EITERS, body, 0)
            plsc.subcore_barrier()

            for cv in range(2):
                g = 2 * p + cv
                if g >= ngroups:
                    continue

                @pl.when(c == cv)
                def _(g=g):
                    pltpu.sync_copy(shared.at[pl.ds(base, _ROWS_PER_TILE), :],
                                    outs[g].at[pl.ds(base, _ROWS_PER_TILE), :])
            if p + 1 < n_pass:
                plsc.subcore_barrier()
    return agg


def _make_pool():
    out_type = [jax.ShapeDtypeStruct((_G, 16), jnp.float32)] * 3
    scratch = [
        pltpu.VMEM((128,), jnp.int32),
        pltpu.VMEM((128, 16), jnp.float32),
        pltpu.VMEM((128, 16), jnp.float32),
        pltpu.VMEM((72, 16), jnp.float32),
        pltpu.VMEM_SHARED((_PG, 16), jnp.float32),
        pltpu.VMEM_SHARED((_PG, 16), jnp.float32),
    ]

    @functools.partial(pl.kernel, out_type=out_type, mesh=_sc_mesh(),
                       scratch_types=scratch,
                       compiler_params=pltpu.CompilerParams(
                           use_tc_tiling_on_sc=False))
    def pool(h0_hbm, h1_hbm, batch_hbm, out0, out1, outc,
             bidx, rows128, ones128, zb, spool, scnt):
        c = lax.axis_index("c")
        s = lax.axis_index("s")

        def fill(i, carry):
            ones128[i, :] = jnp.ones((16,), jnp.float32)
            return carry
        lax.fori_loop(0, 128, fill, 0)

        def zfill(i, carry):
            zb[i, :] = jnp.zeros((16,), jnp.float32)
            return carry
        lax.fori_loop(0, 72, zfill, 0)

        pltpu.sync_copy(zb, spool.at[pl.ds(s * 72, 72), :])
        pltpu.sync_copy(zb, scnt.at[pl.ds(s * 72, 72), :])
        plsc.subcore_barrier()

        def body(j, carry):
            r = s * 49 + j
            pltpu.sync_copy(batch_hbm.at[r], bidx)
            node0 = s * _ROWS_PER_TILE + j * 128
            for cv, tbl in ((0, h0_hbm), (1, h1_hbm)):
                @pl.when(c == cv)
                def _(tbl=tbl):
                    pltpu.sync_copy(tbl.at[pl.ds(node0, 128), :], rows128)
            pltpu.sync_copy(rows128, spool.at[bidx], add=True)

            @pl.when(c == 0)
            def _():
                pltpu.sync_copy(ones128, scnt.at[bidx], add=True)
            return carry
        lax.fori_loop(0, 49, body, 0)
        plsc.subcore_barrier()

        for cv, out in ((0, out0), (1, out1)):
            @pl.when((c == cv) & (s == 0))
            def _(out=out):
                pltpu.sync_copy(spool.at[pl.ds(0, _G), :], out)

        @pl.when((c == 0) & (s == 1))
        def _():
            pltpu.sync_copy(scnt.at[pl.ds(0, _G), :], outc)
    return pool


def _tc_map(aggr_list, h_list, const_list, n_out, compute):
    n_a, n_h = len(aggr_list), len(h_list)

    def kbody(*refs):
        a_refs = refs[:n_a]
        h_refs = refs[n_a:n_a + n_h]
        c_refs = refs[n_a + n_h:n_a + n_h + len(const_list)]
        o_refs = refs[n_a + n_h + len(const_list):]
        A = (jnp.concatenate([r[...] for r in a_refs], axis=1)
             if n_a > 1 else a_refs[0][...])
        H = (jnp.concatenate([r[...] for r in h_refs], axis=1)
             if n_h > 1 else h_refs[0][...])
        res = compute(A, H, [r[...] for r in c_refs])
        for i in range(n_out):
            o_refs[i][...] = res[:, 16 * i:16 * (i + 1)]

    in_specs = ([pl.BlockSpec((_BLK, 16), lambda i: (i, 0))
                 for _ in range(n_a + n_h)] +
                [pl.BlockSpec(cst.shape, lambda i, nd=cst.ndim: (0,) * nd)
                 for cst in const_list])
    out_specs = [pl.BlockSpec((_BLK, 16), lambda i: (i, 0))] * n_out
    out_shape = [jax.ShapeDtypeStruct((_NP, 16), jnp.float32)] * n_out
    return pl.pallas_call(
        kbody, grid=(_NBLKS,), in_specs=in_specs, out_specs=out_specs,
        out_shape=out_shape)(*aggr_list, *h_list, *const_list)


def _ln_relu(y, g, b):
    m = jnp.mean(y, axis=1, keepdims=True)
    yc = y - m
    v = jnp.mean(yc * yc, axis=1, keepdims=True)
    return jnp.maximum(yc * lax.rsqrt(v + 1e-5) * g + b, 0.0)


def _make_mlp(consts):
    def kbody(p0, p1, cnt, *rest):
        c_refs = rest[:len(consts)]
        out = rest[-1]
        wd1, bd1, wd2, bd2, wo, bo = [r[...] for r in c_refs]
        sums = jnp.concatenate([p0[...], p1[...]], axis=1)
        n = jnp.maximum(cnt[:, 0:1], 1.0)
        h = sums / n
        h = jnp.maximum(jnp.dot(h, wd1, preferred_element_type=jnp.float32, precision=lax.Precision.HIGHEST) + bd1, 0.0)
        h = jnp.maximum(jnp.dot(h, wd2, preferred_element_type=jnp.float32, precision=lax.Precision.HIGHEST) + bd2, 0.0)
        out[...] = jnp.dot(h, wo, preferred_element_type=jnp.float32, precision=lax.Precision.HIGHEST) + bo

    in_specs = ([pl.BlockSpec((_G, 16), lambda: (0, 0))] * 3 +
                [pl.BlockSpec(cst.shape, lambda nd=cst.ndim: (0,) * nd)
                 for cst in consts])
    return pl.pallas_call(
        kbody, in_specs=in_specs,
        out_specs=pl.BlockSpec((_G, 1), lambda: (0, 0)),
        out_shape=jax.ShapeDtypeStruct((_G, 1), jnp.float32))


def kernel(x, edge_index, batch, params):
    f32 = jnp.float32

    xp = jnp.zeros((_NP, 16), f32).at[:_N, :_F].set(x)
    pad = jnp.full((_EP - _E,), _N, jnp.int32)
    src2 = jnp.concatenate([edge_index[0], pad]).reshape(_EROWS, 128)
    dst2 = jnp.concatenate([edge_index[1], pad]).reshape(_EROWS, 128)
    batch2 = jnp.concatenate(
        [batch, jnp.full((_NP - _N,), _G, jnp.int32)]).reshape(_NP // 128, 128)

    conv, ln = params["conv"], params["ln"]
    w1rel = jnp.zeros((16, 32), f32).at[:_F, :].set(conv[0]["W_rel"].T)
    w1root = jnp.zeros((16, 32), f32).at[:_F, :].set(conv[0]["W_root"].T)
    c1 = [w1rel, conv[0]["b_rel"][None, :], w1root,
          ln[0]["g"][None, :], ln[0]["b"][None, :]]
    c2 = [conv[1]["W_rel"].T, conv[1]["b_rel"][None, :], conv[1]["W_root"].T,
          ln[1]["g"][None, :], ln[1]["b"][None, :]]
    c3 = [conv[2]["W_rel"].T, conv[2]["b_rel"][None, :], conv[2]["W_root"].T,
          ln[2]["g"][None, :], ln[2]["b"][None, :], conv[3]["W_rel"].T]
    c4 = [conv[3]["b_rel"][None, :], conv[3]["W_root"].T,
          ln[3]["g"][None, :], ln[3]["b"][None, :]]
    dense, outp = params["dense"], params["out"]
    cm = [dense[0]["W"].T, dense[0]["b"][None, :],
          dense[1]["W"].T, dense[1]["b"][None, :],
          outp["W"].T, outp["b"][None, :]]

    agg1, agg2, agg4 = _make_agg(1), _make_agg(2), _make_agg(4)

    (a0,) = agg1(src2, dst2, xp)
    h1 = _tc_map([a0], [xp], c1, 2,
                 lambda A, H, C: _ln_relu(
                     jnp.dot(A, C[0], preferred_element_type=f32, precision=lax.Precision.HIGHEST) + C[1]
                     + jnp.dot(H, C[2], preferred_element_type=f32, precision=lax.Precision.HIGHEST),
                     C[3], C[4]))

    a2 = agg2(src2, dst2, *h1)
    h2 = _tc_map(a2, h1, c2, 4,
                 lambda A, H, C: _ln_relu(
                     jnp.dot(A, C[0], preferred_element_type=f32, precision=lax.Precision.HIGHEST) + C[1]
                     + jnp.dot(H, C[2], preferred_element_type=f32, precision=lax.Precision.HIGHEST),
                     C[3], C[4]))

    a3 = agg4(src2, dst2, *h2)

    def comp3(A, H, C):
        h3 = _ln_relu(jnp.dot(A, C[0], preferred_element_type=f32, precision=lax.Precision.HIGHEST) + C[1]
                      + jnp.dot(H, C[2], preferred_element_type=f32, precision=lax.Precision.HIGHEST),
                      C[3], C[4])
        hm = jnp.dot(h3, C[5], preferred_element_type=f32, precision=lax.Precision.HIGHEST)
        return jnp.concatenate([h3, hm], axis=1)
    h3hm = _tc_map(a3, h2, c3, 6, comp3)
    h3, hm = h3hm[:4], h3hm[4:]

    a4 = agg2(src2, dst2, *hm)
    h4 = _tc_map(a4, h3, c4, 2,
                 lambda A, H, C: _ln_relu(
                     A + C[0] + jnp.dot(H, C[1], preferred_element_type=f32, precision=lax.Precision.HIGHEST),
                     C[2], C[3]))

    p0, p1, cnt = _make_pool()(h4[0], h4[1], batch2)
    return _make_mlp(cm)(p0, p1, cnt, *cm)


# speedup vs baseline: 8.9611x; 1.2107x over previous
import functools

import jax
import jax.numpy as jnp
from jax import lax
from jax.experimental import pallas as pl
from jax.experimental.pallas import tpu as pltpu
from jax.experimental.pallas import tpu_sc as plsc

_N = 100000
_E = 1600000
_F = 5
_G = 1024

_NP = 100352
_ROWS_PER_TILE = 6272
_EROWS = 12544
_EP = _EROWS * 128
_TILE_EROWS = 784
_EITERS = 98
_BLK = 512
_NBLKS = _NP // _BLK
_PG = 1152

@functools.cache
def _sc_mesh():
    return plsc.VectorSubcoreMesh(core_axis_name="c", subcore_axis_name="s")


def _make_agg(ngroups):
    n_pass = (ngroups + 1) // 2
    out_type = [jax.ShapeDtypeStruct((_NP, 16), jnp.float32)] * ngroups
    scratch = [
        [pltpu.VMEM((4, 128), jnp.int32)] * 2,
        [pltpu.VMEM((4, 128), jnp.int32)] * 2,
        [pltpu.VMEM((4, 128, 16), jnp.float32)] * 2,
        pltpu.VMEM((196, 16), jnp.float32),
        pltpu.VMEM_SHARED((_NP, 16), jnp.float32),
        [pltpu.SemaphoreType.DMA] * 2,
        [pltpu.SemaphoreType.DMA] * 2,
        [pltpu.SemaphoreType.DMA] * 2,
    ]

    @functools.partial(pl.kernel, out_type=out_type, mesh=_sc_mesh(),
                       scratch_types=scratch,
                       compiler_params=pltpu.CompilerParams(
                           use_tc_tiling_on_sc=False))
    def agg(src_hbm, dst_hbm, *rest):
        tables = rest[:ngroups]
        outs = rest[ngroups:2 * ngroups]
        (srcs, dsts, rowsb, zbuf, shared, isems, gsems, ssems) = \
            rest[2 * ngroups:]
        c = lax.axis_index("c")
        s = lax.axis_index("s")
        base = s * _ROWS_PER_TILE

        def zb(i, carry):
            zbuf[i, :] = jnp.zeros((16,), jnp.float32)
            return carry
        lax.fori_loop(0, 196, zb, 0)

        for p in range(n_pass):
            def zrow(blk, carry):
                pltpu.sync_copy(zbuf,
                                shared.at[pl.ds(base + blk * 196, 196), :])
                return carry
            lax.fori_loop(0, 32, zrow, 0)
            plsc.subcore_barrier()

            for cv in range(2):
                g = 2 * p + cv
                if g >= ngroups:
                    continue

                @pl.when(c == cv)
                def _(g=g):
                    table = tables[g]

                    def idx_fire(cb, b):
                        r0 = s * _TILE_EROWS + cb * 4
                        pltpu.async_copy(src_hbm.at[pl.ds(r0, 4), :],
                                         srcs[b], isems[b])
                        pltpu.async_copy(dst_hbm.at[pl.ds(r0, 4), :],
                                         dsts[b], isems[b])

                    def idx_wait(b):
                        pltpu.make_async_copy(
                            src_hbm.at[pl.ds(0, 4), :], srcs[b],
                            isems[b]).wait()
                        pltpu.make_async_copy(
                            dst_hbm.at[pl.ds(0, 4), :], dsts[b],
                            isems[b]).wait()

                    def g_fire(b):
                        for k in range(4):
                            pltpu.async_copy(table.at[srcs[b].at[k]],
                                             rowsb[b].at[k], gsems[b])

                    def g_wait(b):
                        for k in range(4):
                            pltpu.make_async_copy(
                                table.at[srcs[b].at[k]], rowsb[b].at[k],
                                gsems[b]).wait()

                    def s_fire(b):
                        for k in range(4):
                            pltpu.async_copy(rowsb[b].at[k],
                                             shared.at[dsts[b].at[k]],
                                             ssems[b], add=True)

                    def s_wait(b):
                        for k in range(4):
                            pltpu.make_async_copy(
                                rowsb[b].at[k], shared.at[dsts[b].at[k]],
                                ssems[b]).wait()

                    pltpu.sync_copy(src_hbm.at[pl.ds(s * _TILE_EROWS, 4), :],
                                    srcs[0])
                    pltpu.sync_copy(dst_hbm.at[pl.ds(s * _TILE_EROWS, 4), :],
                                    dsts[0])
                    g_fire(0)

                    def body(jj, carry):
                        for b in range(2):
                            cb = 2 * jj + b
                            nb = 1 - b
                            if b == 0:
                                @pl.when(jj > 0)
                                def _():
                                    s_wait(nb)
                            else:
                                s_wait(nb)
                            if b == 0:
                                idx_fire(cb + 1, nb)
                            else:
                                @pl.when(jj < _EITERS - 1)
                                def _():
                                    idx_fire(cb + 1, nb)
                            g_wait(b)
                            if b == 0:
                                idx_wait(nb)
                                g_fire(nb)
                            else:
                                @pl.when(jj < _EITERS - 1)
                                def _():
                                    idx_wait(nb)
                                    g_fire(nb)
                            s_fire(b)
                        return carry
                    lax.fori_loop(0, _EITERS, body, 0)
                    s_wait(1)
            plsc.subcore_barrier()

            for cv in range(2):
                g = 2 * p + cv
                if g >= ngroups:
                    continue

                @pl.when(c == cv)
                def _(g=g):
                    pltpu.sync_copy(shared.at[pl.ds(base, _ROWS_PER_TILE), :],
                                    outs[g].at[pl.ds(base, _ROWS_PER_TILE), :])
            if p + 1 < n_pass:
                plsc.subcore_barrier()
    return agg


def _make_pool():
    out_type = [jax.ShapeDtypeStruct((_G, 16), jnp.float32)] * 3
    scratch = [
        pltpu.VMEM((128,), jnp.int32),
        pltpu.VMEM((128, 16), jnp.float32),
        pltpu.VMEM((128, 16), jnp.float32),
        pltpu.VMEM((72, 16), jnp.float32),
        pltpu.VMEM_SHARED((_PG, 16), jnp.float32),
        pltpu.VMEM_SHARED((_PG, 16), jnp.float32),
    ]

    @functools.partial(pl.kernel, out_type=out_type, mesh=_sc_mesh(),
                       scratch_types=scratch,
                       compiler_params=pltpu.CompilerParams(
                           use_tc_tiling_on_sc=False))
    def pool(h0_hbm, h1_hbm, batch_hbm, out0, out1, outc,
             bidx, rows128, ones128, zb, spool, scnt):
        c = lax.axis_index("c")
        s = lax.axis_index("s")

        def fill(i, carry):
            ones128[i, :] = jnp.ones((16,), jnp.float32)
            return carry
        lax.fori_loop(0, 128, fill, 0)

        def zfill(i, carry):
            zb[i, :] = jnp.zeros((16,), jnp.float32)
            return carry
        lax.fori_loop(0, 72, zfill, 0)

        pltpu.sync_copy(zb, spool.at[pl.ds(s * 72, 72), :])
        pltpu.sync_copy(zb, scnt.at[pl.ds(s * 72, 72), :])
        plsc.subcore_barrier()

        def body(j, carry):
            r = s * 49 + j
            pltpu.sync_copy(batch_hbm.at[r], bidx)
            node0 = s * _ROWS_PER_TILE + j * 128
            for cv, tbl in ((0, h0_hbm), (1, h1_hbm)):
                @pl.when(c == cv)
                def _(tbl=tbl):
                    pltpu.sync_copy(tbl.at[pl.ds(node0, 128), :], rows128)
            pltpu.sync_copy(rows128, spool.at[bidx], add=True)

            @pl.when(c == 0)
            def _():
                pltpu.sync_copy(ones128, scnt.at[bidx], add=True)
            return carry
        lax.fori_loop(0, 49, body, 0)
        plsc.subcore_barrier()

        for cv, out in ((0, out0), (1, out1)):
            @pl.when((c == cv) & (s == 0))
            def _(out=out):
                pltpu.sync_copy(spool.at[pl.ds(0, _G), :], out)

        @pl.when((c == 0) & (s == 1))
        def _():
            pltpu.sync_copy(scnt.at[pl.ds(0, _G), :], outc)
    return pool


def _tc_map(aggr_list, h_list, const_list, n_out, compute):
    n_a, n_h = len(aggr_list), len(h_list)

    def kbody(*refs):
        a_refs = refs[:n_a]
        h_refs = refs[n_a:n_a + n_h]
        c_refs = refs[n_a + n_h:n_a + n_h + len(const_list)]
        o_refs = refs[n_a + n_h + len(const_list):]
        A = (jnp.concatenate([r[...] for r in a_refs], axis=1)
             if n_a > 1 else a_refs[0][...])
        H = (jnp.concatenate([r[...] for r in h_refs], axis=1)
             if n_h > 1 else h_refs[0][...])
        res = compute(A, H, [r[...] for r in c_refs])
        for i in range(n_out):
            o_refs[i][...] = res[:, 16 * i:16 * (i + 1)]

    in_specs = ([pl.BlockSpec((_BLK, 16), lambda i: (i, 0))
                 for _ in range(n_a + n_h)] +
                [pl.BlockSpec(cst.shape, lambda i, nd=cst.ndim: (0,) * nd)
                 for cst in const_list])
    out_specs = [pl.BlockSpec((_BLK, 16), lambda i: (i, 0))] * n_out
    out_shape = [jax.ShapeDtypeStruct((_NP, 16), jnp.float32)] * n_out
    return pl.pallas_call(
        kbody, grid=(_NBLKS,), in_specs=in_specs, out_specs=out_specs,
        out_shape=out_shape)(*aggr_list, *h_list, *const_list)


def _ln_relu(y, g, b):
    m = jnp.mean(y, axis=1, keepdims=True)
    yc = y - m
    v = jnp.mean(yc * yc, axis=1, keepdims=True)
    return jnp.maximum(yc * lax.rsqrt(v + 1e-5) * g + b, 0.0)


def _make_mlp(consts):
    def kbody(p0, p1, cnt, *rest):
        c_refs = rest[:len(consts)]
        out = rest[-1]
        wd1, bd1, wd2, bd2, wo, bo = [r[...] for r in c_refs]
        sums = jnp.concatenate([p0[...], p1[...]], axis=1)
        n = jnp.maximum(cnt[:, 0:1], 1.0)
        h = sums / n
        h = jnp.maximum(jnp.dot(h, wd1, preferred_element_type=jnp.float32, precision=lax.Precision.HIGHEST) + bd1, 0.0)
        h = jnp.maximum(jnp.dot(h, wd2, preferred_element_type=jnp.float32, precision=lax.Precision.HIGHEST) + bd2, 0.0)
        out[...] = jnp.dot(h, wo, preferred_element_type=jnp.float32, precision=lax.Precision.HIGHEST) + bo

    in_specs = ([pl.BlockSpec((_G, 16), lambda: (0, 0))] * 3 +
                [pl.BlockSpec(cst.shape, lambda nd=cst.ndim: (0,) * nd)
                 for cst in consts])
    return pl.pallas_call(
        kbody, in_specs=in_specs,
        out_specs=pl.BlockSpec((_G, 1), lambda: (0, 0)),
        out_shape=jax.ShapeDtypeStruct((_G, 1), jnp.float32))


def kernel(x, edge_index, batch, params):
    f32 = jnp.float32

    xp = jnp.zeros((_NP, 16), f32).at[:_N, :_F].set(x)
    pad = jnp.full((_EP - _E,), _N, jnp.int32)
    src2 = jnp.concatenate([edge_index[0], pad]).reshape(_EROWS, 128)
    dst2 = jnp.concatenate([edge_index[1], pad]).reshape(_EROWS, 128)
    batch2 = jnp.concatenate(
        [batch, jnp.full((_NP - _N,), _G, jnp.int32)]).reshape(_NP // 128, 128)

    conv, ln = params["conv"], params["ln"]
    w1rel = jnp.zeros((16, 32), f32).at[:_F, :].set(conv[0]["W_rel"].T)
    w1root = jnp.zeros((16, 32), f32).at[:_F, :].set(conv[0]["W_root"].T)
    c1 = [w1rel, conv[0]["b_rel"][None, :], w1root,
          ln[0]["g"][None, :], ln[0]["b"][None, :]]
    c2 = [conv[1]["W_rel"].T, conv[1]["b_rel"][None, :], conv[1]["W_root"].T,
          ln[1]["g"][None, :], ln[1]["b"][None, :]]
    c3 = [conv[2]["W_rel"].T, conv[2]["b_rel"][None, :], conv[2]["W_root"].T,
          ln[2]["g"][None, :], ln[2]["b"][None, :], conv[3]["W_rel"].T]
    c4 = [conv[3]["b_rel"][None, :], conv[3]["W_root"].T,
          ln[3]["g"][None, :], ln[3]["b"][None, :]]
    dense, outp = params["dense"], params["out"]
    cm = [dense[0]["W"].T, dense[0]["b"][None, :],
          dense[1]["W"].T, dense[1]["b"][None, :],
          outp["W"].T, outp["b"][None, :]]

    agg1, agg2, agg4 = _make_agg(1), _make_agg(2), _make_agg(4)

    (a0,) = agg1(src2, dst2, xp)
    h1 = _tc_map([a0], [xp], c1, 2,
                 lambda A, H, C: _ln_relu(
                     jnp.dot(A, C[0], preferred_element_type=f32, precision=lax.Precision.HIGHEST) + C[1]
                     + jnp.dot(H, C[2], preferred_element_type=f32, precision=lax.Precision.HIGHEST),
                     C[3], C[4]))

    a2 = agg2(src2, dst2, *h1)
    h2 = _tc_map(a2, h1, c2, 4,
                 lambda A, H, C: _ln_relu(
                     jnp.dot(A, C[0], preferred_element_type=f32, precision=lax.Precision.HIGHEST) + C[1]
                     + jnp.dot(H, C[2], preferred_element_type=f32, precision=lax.Precision.HIGHEST),
                     C[3], C[4]))

    a3 = agg4(src2, dst2, *h2)

    def comp3(A, H, C):
        h3 = _ln_relu(jnp.dot(A, C[0], preferred_element_type=f32, precision=lax.Precision.HIGHEST) + C[1]
                      + jnp.dot(H, C[2], preferred_element_type=f32, precision=lax.Precision.HIGHEST),
                      C[3], C[4])
        hm = jnp.dot(h3, C[5], preferred_element_type=f32, precision=lax.Precision.HIGHEST)
        return jnp.concatenate([h3, hm], axis=1)
    h3hm = _tc_map(a3, h2, c3, 6, comp3)
    h3, hm = h3hm[:4], h3hm[4:]

    a4 = agg2(src2, dst2, *hm)
    h4 = _tc_map(a4, h3, c4, 2,
                 lambda A, H, C: _ln_relu(
                     A + C[0] + jnp.dot(H, C[1], preferred_element_type=f32, precision=lax.Precision.HIGHEST),
                     C[2], C[3]))

    p0, p1, cnt = _make_pool()(h4[0], h4[1], batch2)
    return _make_mlp(cm)(p0, p1, cnt, *cm)


# speedup vs baseline: 8.9666x; 1.0006x over previous
import functools

import jax
import jax.numpy as jnp
from jax import lax
from jax.experimental import pallas as pl
from jax.experimental.pallas import tpu as pltpu
from jax.experimental.pallas import tpu_sc as plsc

_N = 100000
_E = 1600000
_F = 5
_G = 1024

_NP = 100352
_ROWS_PER_TILE = 6272
_EROWS = 12544
_EP = _EROWS * 128
_TILE_EROWS = 784
_EITERS = 98
_BLK = 512
_NBLKS = _NP // _BLK
_PG = 1152

@functools.cache
def _sc_mesh():
    return plsc.VectorSubcoreMesh(core_axis_name="c", subcore_axis_name="s")


def _make_agg(ngroups):
    n_pass = (ngroups + 1) // 2
    out_type = [jax.ShapeDtypeStruct((_NP, 16), jnp.float32)] * ngroups
    scratch = [
        [pltpu.VMEM((512,), jnp.int32)] * 2,
        [pltpu.VMEM((512,), jnp.int32)] * 2,
        [pltpu.VMEM((512, 16), jnp.float32)] * 2,
        pltpu.VMEM((196, 16), jnp.float32),
        pltpu.VMEM_SHARED((_NP, 16), jnp.float32),
        [pltpu.SemaphoreType.DMA] * 2,
        [pltpu.SemaphoreType.DMA] * 2,
        [pltpu.SemaphoreType.DMA] * 2,
    ]

    @functools.partial(pl.kernel, out_type=out_type, mesh=_sc_mesh(),
                       scratch_types=scratch,
                       compiler_params=pltpu.CompilerParams(
                           use_tc_tiling_on_sc=False))
    def agg(src_hbm, dst_hbm, *rest):
        tables = rest[:ngroups]
        outs = rest[ngroups:2 * ngroups]
        (srcs, dsts, rowsb, zbuf, shared, isems, gsems, ssems) = \
            rest[2 * ngroups:]
        c = lax.axis_index("c")
        s = lax.axis_index("s")
        base = s * _ROWS_PER_TILE

        def zb(i, carry):
            zbuf[i, :] = jnp.zeros((16,), jnp.float32)
            return carry
        lax.fori_loop(0, 196, zb, 0)

        for p in range(n_pass):
            def zrow(blk, carry):
                pltpu.sync_copy(zbuf,
                                shared.at[pl.ds(base + blk * 196, 196), :])
                return carry
            lax.fori_loop(0, 32, zrow, 0)
            plsc.subcore_barrier()

            for cv in range(2):
                g = 2 * p + cv
                if g >= ngroups:
                    continue

                @pl.when(c == cv)
                def _(g=g):
                    table = tables[g]

                    def idx_fire(cb, b):
                        e0 = (s * _TILE_EROWS + cb * 4) * 128
                        pltpu.async_copy(src_hbm.at[pl.ds(e0, 512)],
                                         srcs[b], isems[b])
                        pltpu.async_copy(dst_hbm.at[pl.ds(e0, 512)],
                                         dsts[b], isems[b])

                    def idx_wait(b):
                        pltpu.make_async_copy(
                            src_hbm.at[pl.ds(0, 512)], srcs[b],
                            isems[b]).wait()
                        pltpu.make_async_copy(
                            dst_hbm.at[pl.ds(0, 512)], dsts[b],
                            isems[b]).wait()

                    def g_fire(b):
                        pltpu.async_copy(table.at[srcs[b]], rowsb[b],
                                         gsems[b])

                    def g_wait(b):
                        pltpu.make_async_copy(table.at[srcs[b]], rowsb[b],
                                              gsems[b]).wait()

                    def s_fire(b):
                        pltpu.async_copy(rowsb[b], shared.at[dsts[b]],
                                         ssems[b], add=True)

                    def s_wait(b):
                        pltpu.make_async_copy(rowsb[b], shared.at[dsts[b]],
                                              ssems[b]).wait()

                    pltpu.sync_copy(
                        src_hbm.at[pl.ds(s * _TILE_EROWS * 128, 512)], srcs[0])
                    pltpu.sync_copy(
                        dst_hbm.at[pl.ds(s * _TILE_EROWS * 128, 512)], dsts[0])
                    g_fire(0)

                    def body(jj, carry):
                        for b in range(2):
                            cb = 2 * jj + b
                            nb = 1 - b
                            if b == 0:
                                @pl.when(jj > 0)
                                def _():
                                    s_wait(nb)
                            else:
                                s_wait(nb)
                            if b == 0:
                                idx_fire(cb + 1, nb)
                            else:
                                @pl.when(jj < _EITERS - 1)
                                def _():
                                    idx_fire(cb + 1, nb)
                            g_wait(b)
                            if b == 0:
                                idx_wait(nb)
                                g_fire(nb)
                            else:
                                @pl.when(jj < _EITERS - 1)
                                def _():
                                    idx_wait(nb)
                                    g_fire(nb)
                            s_fire(b)
                        return carry
                    lax.fori_loop(0, _EITERS, body, 0)
                    s_wait(1)
            plsc.subcore_barrier()

            for cv in range(2):
                g = 2 * p + cv
                if g >= ngroups:
                    continue

                @pl.when(c == cv)
                def _(g=g):
                    pltpu.sync_copy(shared.at[pl.ds(base, _ROWS_PER_TILE), :],
                                    outs[g].at[pl.ds(base, _ROWS_PER_TILE), :])
            if p + 1 < n_pass:
                plsc.subcore_barrier()
    return agg


def _make_pool():
    out_type = [jax.ShapeDtypeStruct((_G, 16), jnp.float32)] * 3
    scratch = [
        pltpu.VMEM((128,), jnp.int32),
        pltpu.VMEM((128, 16), jnp.float32),
        pltpu.VMEM((128, 16), jnp.float32),
        pltpu.VMEM((72, 16), jnp.float32),
        pltpu.VMEM_SHARED((_PG, 16), jnp.float32),
        pltpu.VMEM_SHARED((_PG, 16), jnp.float32),
    ]

    @functools.partial(pl.kernel, out_type=out_type, mesh=_sc_mesh(),
                       scratch_types=scratch,
                       compiler_params=pltpu.CompilerParams(
                           use_tc_tiling_on_sc=False))
    def pool(h0_hbm, h1_hbm, batch_hbm, out0, out1, outc,
             bidx, rows128, ones128, zb, spool, scnt):
        c = lax.axis_index("c")
        s = lax.axis_index("s")

        def fill(i, carry):
            ones128[i, :] = jnp.ones((16,), jnp.float32)
            return carry
        lax.fori_loop(0, 128, fill, 0)

        def zfill(i, carry):
            zb[i, :] = jnp.zeros((16,), jnp.float32)
            return carry
        lax.fori_loop(0, 72, zfill, 0)

        pltpu.sync_copy(zb, spool.at[pl.ds(s * 72, 72), :])
        pltpu.sync_copy(zb, scnt.at[pl.ds(s * 72, 72), :])
        plsc.subcore_barrier()

        def body(j, carry):
            r = s * 49 + j
            pltpu.sync_copy(batch_hbm.at[r], bidx)
            node0 = s * _ROWS_PER_TILE + j * 128
            for cv, tbl in ((0, h0_hbm), (1, h1_hbm)):
                @pl.when(c == cv)
                def _(tbl=tbl):
                    pltpu.sync_copy(tbl.at[pl.ds(node0, 128), :], rows128)
            pltpu.sync_copy(rows128, spool.at[bidx], add=True)

            @pl.when(c == 0)
            def _():
                pltpu.sync_copy(ones128, scnt.at[bidx], add=True)
            return carry
        lax.fori_loop(0, 49, body, 0)
        plsc.subcore_barrier()

        for cv, out in ((0, out0), (1, out1)):
            @pl.when((c == cv) & (s == 0))
            def _(out=out):
                pltpu.sync_copy(spool.at[pl.ds(0, _G), :], out)

        @pl.when((c == 0) & (s == 1))
        def _():
            pltpu.sync_copy(scnt.at[pl.ds(0, _G), :], outc)
    return pool


def _tc_map(aggr_list, h_list, const_list, n_out, compute):
    n_a, n_h = len(aggr_list), len(h_list)

    def kbody(*refs):
        a_refs = refs[:n_a]
        h_refs = refs[n_a:n_a + n_h]
        c_refs = refs[n_a + n_h:n_a + n_h + len(const_list)]
        o_refs = refs[n_a + n_h + len(const_list):]
        A = (jnp.concatenate([r[...] for r in a_refs], axis=1)
             if n_a > 1 else a_refs[0][...])
        H = (jnp.concatenate([r[...] for r in h_refs], axis=1)
             if n_h > 1 else h_refs[0][...])
        res = compute(A, H, [r[...] for r in c_refs])
        for i in range(n_out):
            o_refs[i][...] = res[:, 16 * i:16 * (i + 1)]

    in_specs = ([pl.BlockSpec((_BLK, 16), lambda i: (i, 0))
                 for _ in range(n_a + n_h)] +
                [pl.BlockSpec(cst.shape, lambda i, nd=cst.ndim: (0,) * nd)
                 for cst in const_list])
    out_specs = [pl.BlockSpec((_BLK, 16), lambda i: (i, 0))] * n_out
    out_shape = [jax.ShapeDtypeStruct((_NP, 16), jnp.float32)] * n_out
    return pl.pallas_call(
        kbody, grid=(_NBLKS,), in_specs=in_specs, out_specs=out_specs,
        out_shape=out_shape)(*aggr_list, *h_list, *const_list)


def _ln_relu(y, g, b):
    m = jnp.mean(y, axis=1, keepdims=True)
    yc = y - m
    v = jnp.mean(yc * yc, axis=1, keepdims=True)
    return jnp.maximum(yc * lax.rsqrt(v + 1e-5) * g + b, 0.0)


def _make_mlp(consts):
    def kbody(p0, p1, cnt, *rest):
        c_refs = rest[:len(consts)]
        out = rest[-1]
        wd1, bd1, wd2, bd2, wo, bo = [r[...] for r in c_refs]
        sums = jnp.concatenate([p0[...], p1[...]], axis=1)
        n = jnp.maximum(cnt[:, 0:1], 1.0)
        h = sums / n
        h = jnp.maximum(jnp.dot(h, wd1, preferred_element_type=jnp.float32, precision=lax.Precision.HIGHEST) + bd1, 0.0)
        h = jnp.maximum(jnp.dot(h, wd2, preferred_element_type=jnp.float32, precision=lax.Precision.HIGHEST) + bd2, 0.0)
        out[...] = jnp.dot(h, wo, preferred_element_type=jnp.float32, precision=lax.Precision.HIGHEST) + bo

    in_specs = ([pl.BlockSpec((_G, 16), lambda: (0, 0))] * 3 +
                [pl.BlockSpec(cst.shape, lambda nd=cst.ndim: (0,) * nd)
                 for cst in consts])
    return pl.pallas_call(
        kbody, in_specs=in_specs,
        out_specs=pl.BlockSpec((_G, 1), lambda: (0, 0)),
        out_shape=jax.ShapeDtypeStruct((_G, 1), jnp.float32))


def kernel(x, edge_index, batch, params):
    f32 = jnp.float32

    xp = jnp.zeros((_NP, 16), f32).at[:_N, :_F].set(x)
    pad = jnp.full((_EP - _E,), _N, jnp.int32)
    src2 = jnp.concatenate([edge_index[0], pad])
    dst2 = jnp.concatenate([edge_index[1], pad])
    batch2 = jnp.concatenate(
        [batch, jnp.full((_NP - _N,), _G, jnp.int32)]).reshape(_NP // 128, 128)

    conv, ln = params["conv"], params["ln"]
    w1rel = jnp.zeros((16, 32), f32).at[:_F, :].set(conv[0]["W_rel"].T)
    w1root = jnp.zeros((16, 32), f32).at[:_F, :].set(conv[0]["W_root"].T)
    c1 = [w1rel, conv[0]["b_rel"][None, :], w1root,
          ln[0]["g"][None, :], ln[0]["b"][None, :]]
    c2 = [conv[1]["W_rel"].T, conv[1]["b_rel"][None, :], conv[1]["W_root"].T,
          ln[1]["g"][None, :], ln[1]["b"][None, :]]
    c3 = [conv[2]["W_rel"].T, conv[2]["b_rel"][None, :], conv[2]["W_root"].T,
          ln[2]["g"][None, :], ln[2]["b"][None, :], conv[3]["W_rel"].T]
    c4 = [conv[3]["b_rel"][None, :], conv[3]["W_root"].T,
          ln[3]["g"][None, :], ln[3]["b"][None, :]]
    dense, outp = params["dense"], params["out"]
    cm = [dense[0]["W"].T, dense[0]["b"][None, :],
          dense[1]["W"].T, dense[1]["b"][None, :],
          outp["W"].T, outp["b"][None, :]]

    agg1, agg2, agg4 = _make_agg(1), _make_agg(2), _make_agg(4)

    (a0,) = agg1(src2, dst2, xp)
    h1 = _tc_map([a0], [xp], c1, 2,
                 lambda A, H, C: _ln_relu(
                     jnp.dot(A, C[0], preferred_element_type=f32, precision=lax.Precision.HIGHEST) + C[1]
                     + jnp.dot(H, C[2], preferred_element_type=f32, precision=lax.Precision.HIGHEST),
                     C[3], C[4]))

    a2 = agg2(src2, dst2, *h1)
    h2 = _tc_map(a2, h1, c2, 4,
                 lambda A, H, C: _ln_relu(
                     jnp.dot(A, C[0], preferred_element_type=f32, precision=lax.Precision.HIGHEST) + C[1]
                     + jnp.dot(H, C[2], preferred_element_type=f32, precision=lax.Precision.HIGHEST),
                     C[3], C[4]))

    a3 = agg4(src2, dst2, *h2)

    def comp3(A, H, C):
        h3 = _ln_relu(jnp.dot(A, C[0], preferred_element_type=f32, precision=lax.Precision.HIGHEST) + C[1]
                      + jnp.dot(H, C[2], preferred_element_type=f32, precision=lax.Precision.HIGHEST),
                      C[3], C[4])
        hm = jnp.dot(h3, C[5], preferred_element_type=f32, precision=lax.Precision.HIGHEST)
        return jnp.concatenate([h3, hm], axis=1)
    h3hm = _tc_map(a3, h2, c3, 6, comp3)
    h3, hm = h3hm[:4], h3hm[4:]

    a4 = agg2(src2, dst2, *hm)
    h4 = _tc_map(a4, h3, c4, 2,
                 lambda A, H, C: _ln_relu(
                     A + C[0] + jnp.dot(H, C[1], preferred_element_type=f32, precision=lax.Precision.HIGHEST),
                     C[2], C[3]))

    p0, p1, cnt = _make_pool()(h4[0], h4[1], batch2)
    return _make_mlp(cm)(p0, p1, cnt, *cm)
